# Initial kernel scaffold; baseline (speedup 1.0000x reference)
import jax
import jax.numpy as jnp
from jax.experimental import pallas as pl


def kernel(x, biases, weight_h, weight_v):
    raise NotImplementedError("write your pallas kernel here")


# speedup vs baseline: 899.4748x; 899.4748x over previous
import functools

import jax
import jax.numpy as jnp
from jax import lax
from jax.experimental import pallas as pl
from jax.experimental.pallas import tpu as pltpu
from jax.experimental.pallas import tpu_sc as plsc

H = 224
W = 224
L = 4
B = 1024
LANES = 16
NC = 2
NS = 16
NW = NC * NS
SPW = B // NW
R = 28
NCHUNK = H // R
NBLK = W // LANES
HW = H * W

_mesh = plsc.VectorSubcoreMesh(
    core_axis_name="c", subcore_axis_name="s", num_cores=NC, num_subcores=NS
)


@functools.partial(
    pl.kernel,
    out_type=jax.ShapeDtypeStruct((3, B, LANES), jnp.float32),
    mesh=_mesh,
    scratch_types=[
        pltpu.VMEM((R * W * L,), jnp.float32),
        pltpu.VMEM(((R + 1) * W,), jnp.int32),
        pltpu.VMEM(((R + 1) * W,), jnp.int32),
        pltpu.VMEM((SPW, LANES), jnp.float32),
        pltpu.VMEM((SPW, LANES), jnp.float32),
        pltpu.VMEM((SPW, LANES), jnp.float32),
        pltpu.SemaphoreType.DMA,
        pltpu.SemaphoreType.DMA,
    ],
    compiler_params=pltpu.CompilerParams(needs_layout_passes=False),
)
def _ebm_sc(x_hbm, bias_hbm, out_hbm, bias_v, xb0, xb1, pb, ph, pv, sem0, sem1):
    wid = lax.axis_index("s") * NC + lax.axis_index("c")
    sbase = wid * SPW

    lane = lax.iota(jnp.int32, LANES)
    iota4 = lane * L
    ones = jnp.full((LANES,), 1.0, jnp.float32)
    zeros = jnp.zeros((LANES,), jnp.float32)
    mask15 = jnp.where(lane < LANES - 1, 1.0, 0.0).astype(jnp.float32)

    def init_body(s, _):
        pb[s] = zeros
        ph[s] = zeros
        pv[s] = zeros
        return 0

    lax.fori_loop(0, SPW, init_body, 0)

    def issue(s, r0, nr, buf, sem):
        pltpu.async_copy(
            x_hbm.at[pl.ds((sbase + s) * HW + r0 * W, nr * W)],
            buf.at[pl.ds(0, nr * W)],
            sem,
        )

    def process(buf, s):
        def row_body(r, carry):
            xrow, ba, ha, va = carry
            rowvec = iota4 + r * (W * L)
            newrow = []
            for c in range(NBLK):
                off = r * W + c * LANES
                xr = xrow[c]
                xd = buf[pl.ds(off + W, LANES)]
                xr1 = buf[pl.ds(off + 1, LANES)]
                if c == 0:
                    idx = xr + rowvec
                else:
                    idx = (xr + c * (LANES * L)) + rowvec
                bv = plsc.load_gather(bias_v, [idx])
                ba = ba + bv
                hmask = mask15 if c == NBLK - 1 else ones
                ha = ha + jnp.where(xr == xr1, hmask, zeros)
                va = va + jnp.where(xr == xd, ones, zeros)
                newrow.append(xd)
            return (tuple(newrow), ba, ha, va)

        xrow0 = tuple(buf[pl.ds(c * LANES, LANES)] for c in range(NBLK))
        _, ba, ha, va = lax.fori_loop(
            0, R, row_body, (xrow0, zeros, zeros, zeros)
        )
        plsc.addupdate(pb.at[s], ba)
        plsc.addupdate(ph.at[s], ha)
        plsc.addupdate(pv.at[s], va)

    def run_chunk(k, nr):
        r0 = k * R
        issue(0, r0, nr, xb0, sem0)
        pltpu.sync_copy(bias_hbm.at[pl.ds(r0 * (W * L), R * W * L)], bias_v)

        def pair_body(i, _):
            for b in range(2):
                buf, sem = (xb0, sem0) if b == 0 else (xb1, sem1)
                obuf, osem = (xb1, sem1) if b == 0 else (xb0, sem0)
                s = 2 * i + b
                pltpu.make_async_copy(
                    x_hbm.at[pl.ds(0, nr * W)], buf.at[pl.ds(0, nr * W)], sem
                ).wait()

                @pl.when(s + 1 < SPW)
                def _():
                    issue(s + 1, r0, nr, obuf, osem)

                process(buf, s)
            return 0

        lax.fori_loop(0, SPW // 2, pair_body, 0)

    lax.fori_loop(0, NCHUNK - 1, lambda k, _: (run_chunk(k, R + 1), 0)[1], 0)

    neg = jnp.full((LANES,), -1, jnp.int32)
    for c in range(NBLK):
        xb0[pl.ds(R * W + c * LANES, LANES)] = neg
        xb1[pl.ds(R * W + c * LANES, LANES)] = neg
    run_chunk(NCHUNK - 1, R)

    pltpu.sync_copy(pb, out_hbm.at[0, pl.ds(sbase, SPW)])
    pltpu.sync_copy(ph, out_hbm.at[1, pl.ds(sbase, SPW)])
    pltpu.sync_copy(pv, out_hbm.at[2, pl.ds(sbase, SPW)])


def kernel(x, biases, weight_h, weight_v):
    xf = x.reshape(-1)
    bf = biases.reshape(-1)
    parts = _ebm_sc(xf, bf)
    sums = parts.sum(axis=-1)
    return -(sums[0] + weight_h * sums[1] + weight_v * sums[2])
